# baseline (speedup 1.0000x reference)
import functools

import jax
import jax.numpy as jnp
from jax import lax
from jax.experimental import pallas as pl
from jax.experimental.pallas import tpu as pltpu
from jax.experimental.pallas import tpu_sc as plsc

_L = 16
_K = 25
_NEG_INF = float("-inf")
_CHUNK = 8192
_SUBS = 16
_VPS = _CHUNK // (_SUBS * _L)
_CAND = 4096
_OUTW = 32


@functools.cache
def _sc_collect(n_elems):
    info = plsc.get_sparse_core_info()
    nc, ns = info.num_cores, info.num_subcores
    nw = nc * ns
    slice_len = n_elems // nw
    chunks = slice_len // _CHUNK
    assert slice_len % _CHUNK == 0

    mesh = plsc.VectorSubcoreMesh(core_axis_name="c", subcore_axis_name="s")

    @functools.partial(
        pl.kernel,
        out_type=(
            jax.ShapeDtypeStruct((nw * _OUTW,), jnp.float32),
            jax.ShapeDtypeStruct((nw * _OUTW,), jnp.float32),
        ),
        mesh=mesh,
        compiler_params=pltpu.CompilerParams(needs_layout_passes=False),
        scratch_types=[
            pltpu.VMEM((_CHUNK,), jnp.float32),
            pltpu.VMEM((_CHUNK,), jnp.float32),
            pltpu.VMEM((_CHUNK,), jnp.float32),
            pltpu.VMEM((_CHUNK,), jnp.float32),
            pltpu.VMEM((_CAND,), jnp.float32),
            pltpu.VMEM((_CAND,), jnp.float32),
            pltpu.VMEM((_OUTW,), jnp.float32),
            pltpu.VMEM((_OUTW,), jnp.float32),
            pltpu.SemaphoreType.DMA,
            pltpu.SemaphoreType.DMA,
        ],
    )
    def collect(x_hbm, t_hbm, outn_hbm, outp_hbm,
                xb0, tb0, xb1, tb1, cna, cnb, oa, ob, sem0, sem1):
        ninf = jnp.full((_L,), _NEG_INF, jnp.float32)
        zc = jnp.zeros((_L,), jnp.int32)
        iot = lax.iota(jnp.int32, _L)
        wid = lax.axis_index("s") * nc + lax.axis_index("c")
        base = wid * slice_len

        def initb(j, _):
            cna[pl.ds(j * _L, _L)] = ninf
            cnb[pl.ds(j * _L, _L)] = ninf
            return 0

        lax.fori_loop(0, _CAND // _L, initb, 0)

        def process_chunk(xb, tb, carry):
            def sub_body(s, scarry):
                (cnta, cntb, taua, taub, m1a, m2a, m1b, m2b) = scarry

                def grp_body(g, gc):
                    (cnta, cntb, m1a, m2a, m1b, m2b) = gc
                    gbase = (s * 8 + g) * 4 * _L
                    for j in range(4):
                        idx = gbase + j * _L
                        xv = xb[pl.ds(idx, _L)]
                        tv = tb[pl.ds(idx, _L)]
                        isneg = tv < 0.5
                        a = jnp.where(isneg, xv, ninf)
                        b = jnp.where(isneg, ninf, -xv)
                        if j == 0:
                            m2a = jnp.maximum(m2a, jnp.minimum(m1a, a))
                            m1a = jnp.maximum(m1a, a)
                            m2b = jnp.maximum(m2b, jnp.minimum(m1b, b))
                            m1b = jnp.maximum(m1b, b)
                        ka = a > taua
                        kb = b > taub
                        sa, _ = plsc.sort_key_val(
                            jnp.where(ka, a, ninf), a, descending=True)
                        sb, _ = plsc.sort_key_val(
                            jnp.where(kb, b, ninf), b, descending=True)
                        cna[pl.ds(jnp.minimum(cnta, _CAND - _L), _L)] = sa
                        cnb[pl.ds(jnp.minimum(cntb, _CAND - _L), _L)] = sb
                        cnta = cnta + plsc.all_reduce_population_count(ka)[0]
                        cntb = cntb + plsc.all_reduce_population_count(kb)[0]
                    return (cnta, cntb, m1a, m2a, m1b, m2b)

                (cnta, cntb, m1a, m2a, m1b, m2b) = lax.fori_loop(
                    0, 8, grp_body, (cnta, cntb, m1a, m2a, m1b, m2b))
                taua = jnp.min(m2a)
                taub = jnp.min(m2b)
                return (cnta, cntb, taua, taub, m1a, m2a, m1b, m2b)

            return lax.fori_loop(0, _SUBS, sub_body, carry)

        def mk(ref, c, buf, sem):
            return pltpu.make_async_copy(
                ref.at[pl.ds(base + c * _CHUNK, _CHUNK)], buf, sem)

        mk(x_hbm, 0, xb0, sem0).start()
        mk(t_hbm, 0, tb0, sem0).start()
        mk(x_hbm, 1, xb1, sem1).start()
        mk(t_hbm, 1, tb1, sem1).start()

        def pair_body(g, carry):
            c0 = 2 * g
            c1 = 2 * g + 1
            mk(x_hbm, c0, xb0, sem0).wait()
            mk(t_hbm, c0, tb0, sem0).wait()
            carry = process_chunk(xb0, tb0, carry)

            @pl.when(c0 + 2 < chunks)
            def _():
                mk(x_hbm, c0 + 2, xb0, sem0).start()
                mk(t_hbm, c0 + 2, tb0, sem0).start()

            mk(x_hbm, c1, xb1, sem1).wait()
            mk(t_hbm, c1, tb1, sem1).wait()
            carry = process_chunk(xb1, tb1, carry)

            @pl.when(c1 + 2 < chunks)
            def _():
                mk(x_hbm, c1 + 2, xb1, sem1).start()
                mk(t_hbm, c1 + 2, tb1, sem1).start()

            return carry

        carry = lax.fori_loop(
            0, chunks // 2, pair_body,
            (jnp.int32(0), jnp.int32(0),
             jnp.float32(_NEG_INF), jnp.float32(_NEG_INF),
             ninf, ninf, ninf, ninf))
        cnta, cntb = carry[0], carry[1]

        def select25(cref, cnt):
            nv = (jnp.minimum(cnt, _CAND) + (_L - 1)) // _L

            def kbody(k, kc):
                filled, o0, o1 = kc

                def smax(j, m):
                    return jnp.maximum(m, cref[pl.ds(j * _L, _L)])

                v = jnp.max(lax.fori_loop(0, nv, smax, ninf))

                def srm(j, cacc):
                    vr = cref[pl.ds(j * _L, _L)]
                    eq = vr == v
                    cref[pl.ds(j * _L, _L)] = jnp.where(eq, ninf, vr)
                    return cacc + plsc.all_reduce_population_count(eq)

                cvec = lax.fori_loop(0, nv, srm, zc)
                take = jnp.minimum(cvec[0], _K - filled)
                lo = filled
                hi = filled + take
                o0 = jnp.where((iot >= lo) & (iot < hi), v, o0)
                o1 = jnp.where(((iot + _L) >= lo) & ((iot + _L) < hi), v, o1)
                return (filled + take, o0, o1)

            _, o0, o1 = lax.fori_loop(0, _K, kbody, (jnp.int32(0), ninf, ninf))
            return o0, o1

        o0, o1 = select25(cna, cnta)
        oa[pl.ds(0, _L)] = o0
        oa[pl.ds(_L, _L)] = o1
        p0, p1 = select25(cnb, cntb)
        ob[pl.ds(0, _L)] = p0
        ob[pl.ds(_L, _L)] = p1
        pltpu.sync_copy(oa, outn_hbm.at[pl.ds(wid * _OUTW, _OUTW)])
        pltpu.sync_copy(ob, outp_hbm.at[pl.ds(wid * _OUTW, _OUTW)])

    return collect


def _merge_body(nref, pref, oref):
    lane = lax.broadcasted_iota(jnp.int32, (1, 128), 1)

    def select25_tc(arr0):
        def kbody(k, kc):
            filled, out, arr = kc
            v = jnp.max(arr)
            eq = arr == v
            c = jnp.sum(eq.astype(jnp.int32))
            take = jnp.minimum(c, _K - filled)
            out = jnp.where((lane >= filled) & (lane < filled + take), v, out)
            arr = jnp.where(eq, _NEG_INF, arr)
            return (filled + take, out, arr)

        _, out, _ = lax.fori_loop(
            0, _K, kbody,
            (jnp.int32(0), jnp.full((1, 128), _NEG_INF, jnp.float32), arr0))
        return out

    m25 = lane < _K
    seln = select25_tc(nref[...])
    selp = select25_tc(pref[...])
    pn = jax.nn.sigmoid(seln)
    tn = jnp.maximum(jnp.log(1.0 - pn), -100.0)
    neg_loss = -0.5 * jnp.sum(jnp.where(m25, tn, 0.0)) / _K
    pp = jax.nn.sigmoid(-selp)
    tp = jnp.maximum(jnp.log(pp), -100.0)
    pos_loss = -0.5 * jnp.sum(jnp.where(m25, tp, 0.0)) / _K
    oref[...] = jnp.where(lane == 0, pos_loss,
                          jnp.where(lane == 1, neg_loss, 0.0))


def kernel(font_output, font_target, use_hard_mining):
    x = font_output.reshape(-1)
    t = font_target.reshape(-1)
    n = x.shape[0]

    def hard(_):
        negc, posc = _sc_collect(n)(x, t)
        out = pl.pallas_call(
            _merge_body,
            out_shape=jax.ShapeDtypeStruct((1, 128), jnp.float32),
        )(negc.reshape(8, -1), posc.reshape(8, -1))
        return out[0, 0], out[0, 1]

    def soft(_):
        p = jax.nn.sigmoid(x)
        pos_mask = t == 1
        neg_mask = t == 0
        logp = jnp.clip(jnp.log(p), -100.0, None)
        log1mp = jnp.clip(jnp.log(1.0 - p), -100.0, None)
        pos_loss = 0.5 * jnp.sum(jnp.where(pos_mask, -logp, 0.0)) / jnp.sum(pos_mask)
        neg_loss = 0.5 * jnp.sum(jnp.where(neg_mask, -log1mp, 0.0)) / jnp.sum(neg_mask)
        return pos_loss, neg_loss

    pos_loss, neg_loss = lax.cond(use_hard_mining != 0, hard, soft, operand=None)
    return (pos_loss + neg_loss, pos_loss, neg_loss)


# speedup vs baseline: 1.4119x; 1.4119x over previous
import functools

import jax
import jax.numpy as jnp
from jax import lax
from jax.experimental import pallas as pl
from jax.experimental.pallas import tpu as pltpu
from jax.experimental.pallas import tpu_sc as plsc

_L = 16
_K = 25
_NEG_INF = float("-inf")
_CHUNK = 8192
_SUBS = 16
_VPS = _CHUNK // (_SUBS * _L)
_CAND = 4096
_OUTW = 32


@functools.cache
def _sc_collect(n_elems):
    info = plsc.get_sparse_core_info()
    nc, ns = info.num_cores, info.num_subcores
    nw = nc * ns
    slice_len = n_elems // nw
    chunks = slice_len // _CHUNK
    assert slice_len % _CHUNK == 0

    mesh = plsc.VectorSubcoreMesh(core_axis_name="c", subcore_axis_name="s")

    @functools.partial(
        pl.kernel,
        out_type=(
            jax.ShapeDtypeStruct((nw * _OUTW,), jnp.float32),
            jax.ShapeDtypeStruct((nw * _OUTW,), jnp.float32),
        ),
        mesh=mesh,
        compiler_params=pltpu.CompilerParams(needs_layout_passes=False),
        scratch_types=[
            pltpu.VMEM((_CHUNK,), jnp.float32),
            pltpu.VMEM((_CHUNK,), jnp.float32),
            pltpu.VMEM((_CHUNK,), jnp.float32),
            pltpu.VMEM((_CHUNK,), jnp.float32),
            pltpu.VMEM((_CAND,), jnp.float32),
            pltpu.VMEM((_CAND,), jnp.float32),
            pltpu.VMEM((4 * _L,), jnp.float32),
            pltpu.VMEM((2 * _L,), jnp.int32),
            pltpu.VMEM((_OUTW,), jnp.float32),
            pltpu.VMEM((_OUTW,), jnp.float32),
            pltpu.SemaphoreType.DMA,
            pltpu.SemaphoreType.DMA,
        ],
    )
    def collect(x_hbm, t_hbm, outn_hbm, outp_hbm,
                xb0, tb0, xb1, tb1, cna, cnb, trk, cnts, oa, ob, sem0, sem1):
        ninf = jnp.full((_L,), _NEG_INF, jnp.float32)
        zc = jnp.zeros((_L,), jnp.int32)
        iot = lax.iota(jnp.int32, _L)
        wid = lax.axis_index("s") * nc + lax.axis_index("c")
        base = wid * slice_len

        def initb(j, _):
            cna[pl.ds(j * _L, _L)] = ninf
            cnb[pl.ds(j * _L, _L)] = ninf
            return 0

        lax.fori_loop(0, _CAND // _L, initb, 0)
        for j in range(4):
            trk[pl.ds(j * _L, _L)] = ninf
        cnts[pl.ds(0, _L)] = zc
        cnts[pl.ds(_L, _L)] = zc

        def merge_top32(off, gmd):
            hi = trk[pl.ds(off, _L)]
            nh = jnp.maximum(hi, gmd)
            sp = jnp.minimum(hi, gmd)
            nh, _ = plsc.sort_key_val(nh, nh, descending=False)
            spd, _ = plsc.sort_key_val(sp, sp, descending=True)
            lo_ = trk[pl.ds(off + _L, _L)]
            nl = jnp.maximum(lo_, spd)
            nl, _ = plsc.sort_key_val(nl, nl, descending=False)
            trk[pl.ds(off, _L)] = nh
            trk[pl.ds(off + _L, _L)] = nl

        def process_chunk(xb, tb):
            def sub_body(s, _):
                taua = trk[pl.ds(_L, _L)][0]
                taub = trk[pl.ds(3 * _L, _L)][0]
                ntaub = -taub

                def grp_body(g, __):
                    gbase = (s * 8 + g) * 4 * _L
                    xvs = [xb[pl.ds(gbase + j * _L, _L)] for j in range(4)]
                    mx = jnp.maximum(jnp.maximum(xvs[0], xvs[1]),
                                     jnp.maximum(xvs[2], xvs[3]))
                    mn = jnp.minimum(jnp.minimum(xvs[0], xvs[1]),
                                     jnp.minimum(xvs[2], xvs[3]))
                    hit = (mx > taua) | (mn < ntaub)
                    nhit = plsc.all_reduce_population_count(hit)[0]

                    @pl.when(nhit > 0)
                    def _():
                        cnta = cnts[pl.ds(0, _L)][0]
                        cntb = cnts[pl.ds(_L, _L)][0]
                        ga = ninf
                        gb = ninf
                        for j in range(4):
                            xv = xvs[j]
                            tv = tb[pl.ds(gbase + j * _L, _L)]
                            isneg = tv < 0.5
                            a = jnp.where(isneg, xv, ninf)
                            b = jnp.where(isneg, ninf, -xv)
                            ka = a > taua
                            kb = b > taub
                            ma = jnp.where(ka, a, ninf)
                            mb = jnp.where(kb, b, ninf)
                            sa, _ = plsc.sort_key_val(ma, ma, descending=True)
                            sb, _ = plsc.sort_key_val(mb, mb, descending=True)
                            cna[pl.ds(jnp.minimum(cnta, _CAND - _L), _L)] = sa
                            cnb[pl.ds(jnp.minimum(cntb, _CAND - _L), _L)] = sb
                            cnta = cnta + plsc.all_reduce_population_count(ka)[0]
                            cntb = cntb + plsc.all_reduce_population_count(kb)[0]
                            ga = jnp.maximum(ga, ma)
                            gb = jnp.maximum(gb, mb)
                        cnts[pl.ds(0, _L)] = zc + cnta
                        cnts[pl.ds(_L, _L)] = zc + cntb
                        gad, _ = plsc.sort_key_val(ga, ga, descending=True)
                        gbd, _ = plsc.sort_key_val(gb, gb, descending=True)
                        merge_top32(0, gad)
                        merge_top32(2 * _L, gbd)

                    return 0

                lax.fori_loop(0, 8, grp_body, 0)
                return 0

            lax.fori_loop(0, _SUBS, sub_body, 0)

        def mk(ref, c, buf, sem):
            return pltpu.make_async_copy(
                ref.at[pl.ds(base + c * _CHUNK, _CHUNK)], buf, sem)

        mk(x_hbm, 0, xb0, sem0).start()
        mk(t_hbm, 0, tb0, sem0).start()
        mk(x_hbm, 1, xb1, sem1).start()
        mk(t_hbm, 1, tb1, sem1).start()

        def pair_body(g, _):
            c0 = 2 * g
            c1 = 2 * g + 1
            mk(x_hbm, c0, xb0, sem0).wait()
            mk(t_hbm, c0, tb0, sem0).wait()
            process_chunk(xb0, tb0)

            @pl.when(c0 + 2 < chunks)
            def _():
                mk(x_hbm, c0 + 2, xb0, sem0).start()
                mk(t_hbm, c0 + 2, tb0, sem0).start()

            mk(x_hbm, c1, xb1, sem1).wait()
            mk(t_hbm, c1, tb1, sem1).wait()
            process_chunk(xb1, tb1)

            @pl.when(c1 + 2 < chunks)
            def _():
                mk(x_hbm, c1 + 2, xb1, sem1).start()
                mk(t_hbm, c1 + 2, tb1, sem1).start()

            return 0

        lax.fori_loop(0, chunks // 2, pair_body, 0)
        cnta = cnts[pl.ds(0, _L)][0]
        cntb = cnts[pl.ds(_L, _L)][0]

        def select25(cref, cnt):
            nv = (jnp.minimum(cnt, _CAND) + (_L - 1)) // _L

            def kbody(k, kc):
                filled, o0, o1 = kc

                def smax(j, m):
                    return jnp.maximum(m, cref[pl.ds(j * _L, _L)])

                v = jnp.max(lax.fori_loop(0, nv, smax, ninf))

                def srm(j, cacc):
                    vr = cref[pl.ds(j * _L, _L)]
                    eq = vr == v
                    cref[pl.ds(j * _L, _L)] = jnp.where(eq, ninf, vr)
                    return cacc + plsc.all_reduce_population_count(eq)

                cvec = lax.fori_loop(0, nv, srm, zc)
                take = jnp.minimum(cvec[0], _K - filled)
                lo = filled
                hi = filled + take
                o0 = jnp.where((iot >= lo) & (iot < hi), v, o0)
                o1 = jnp.where(((iot + _L) >= lo) & ((iot + _L) < hi), v, o1)
                return (filled + take, o0, o1)

            _, o0, o1 = lax.fori_loop(0, _K, kbody, (jnp.int32(0), ninf, ninf))
            return o0, o1

        o0, o1 = select25(cna, cnta)
        oa[pl.ds(0, _L)] = o0
        oa[pl.ds(_L, _L)] = o1
        p0, p1 = select25(cnb, cntb)
        ob[pl.ds(0, _L)] = p0
        ob[pl.ds(_L, _L)] = p1
        pltpu.sync_copy(oa, outn_hbm.at[pl.ds(wid * _OUTW, _OUTW)])
        pltpu.sync_copy(ob, outp_hbm.at[pl.ds(wid * _OUTW, _OUTW)])

    return collect


def _merge_body(nref, pref, oref):
    lane = lax.broadcasted_iota(jnp.int32, (1, 128), 1)

    def select25_tc(arr0):
        def kbody(k, kc):
            filled, out, arr = kc
            v = jnp.max(arr)
            eq = arr == v
            c = jnp.sum(eq.astype(jnp.int32))
            take = jnp.minimum(c, _K - filled)
            out = jnp.where((lane >= filled) & (lane < filled + take), v, out)
            arr = jnp.where(eq, _NEG_INF, arr)
            return (filled + take, out, arr)

        _, out, _ = lax.fori_loop(
            0, _K, kbody,
            (jnp.int32(0), jnp.full((1, 128), _NEG_INF, jnp.float32), arr0))
        return out

    m25 = lane < _K
    seln = select25_tc(nref[...])
    selp = select25_tc(pref[...])
    pn = jax.nn.sigmoid(seln)
    tn = jnp.maximum(jnp.log(1.0 - pn), -100.0)
    neg_loss = -0.5 * jnp.sum(jnp.where(m25, tn, 0.0)) / _K
    pp = jax.nn.sigmoid(-selp)
    tp = jnp.maximum(jnp.log(pp), -100.0)
    pos_loss = -0.5 * jnp.sum(jnp.where(m25, tp, 0.0)) / _K
    oref[...] = jnp.where(lane == 0, pos_loss,
                          jnp.where(lane == 1, neg_loss, 0.0))


def kernel(font_output, font_target, use_hard_mining):
    x = font_output.reshape(-1)
    t = font_target.reshape(-1)
    n = x.shape[0]

    def hard(_):
        negc, posc = _sc_collect(n)(x, t)
        out = pl.pallas_call(
            _merge_body,
            out_shape=jax.ShapeDtypeStruct((1, 128), jnp.float32),
        )(negc.reshape(8, -1), posc.reshape(8, -1))
        return out[0, 0], out[0, 1]

    def soft(_):
        p = jax.nn.sigmoid(x)
        pos_mask = t == 1
        neg_mask = t == 0
        logp = jnp.clip(jnp.log(p), -100.0, None)
        log1mp = jnp.clip(jnp.log(1.0 - p), -100.0, None)
        pos_loss = 0.5 * jnp.sum(jnp.where(pos_mask, -logp, 0.0)) / jnp.sum(pos_mask)
        neg_loss = 0.5 * jnp.sum(jnp.where(neg_mask, -log1mp, 0.0)) / jnp.sum(neg_mask)
        return pos_loss, neg_loss

    pos_loss, neg_loss = lax.cond(use_hard_mining != 0, hard, soft, operand=None)
    return (pos_loss + neg_loss, pos_loss, neg_loss)


# speedup vs baseline: 1.4892x; 1.0547x over previous
import functools

import jax
import jax.numpy as jnp
from jax import lax
from jax.experimental import pallas as pl
from jax.experimental.pallas import tpu as pltpu
from jax.experimental.pallas import tpu_sc as plsc

_L = 16
_K = 25
_NEG_INF = float("-inf")
_CHUNK = 8192
_SUBS = 16
_VPS = _CHUNK // (_SUBS * _L)
_CAND = 4096
_OUTW = 32


@functools.cache
def _sc_collect(n_elems):
    info = plsc.get_sparse_core_info()
    nc, ns = info.num_cores, info.num_subcores
    nw = nc * ns
    slice_len = n_elems // nw
    chunks = slice_len // _CHUNK
    assert slice_len % _CHUNK == 0

    mesh = plsc.VectorSubcoreMesh(core_axis_name="c", subcore_axis_name="s")

    @functools.partial(
        pl.kernel,
        out_type=(
            jax.ShapeDtypeStruct((nw * _OUTW,), jnp.float32),
            jax.ShapeDtypeStruct((nw * _OUTW,), jnp.float32),
        ),
        mesh=mesh,
        compiler_params=pltpu.CompilerParams(needs_layout_passes=False),
        scratch_types=[
            pltpu.VMEM((_CHUNK,), jnp.float32),
            pltpu.VMEM((_CHUNK,), jnp.float32),
            pltpu.VMEM((_CHUNK,), jnp.float32),
            pltpu.VMEM((_CHUNK,), jnp.float32),
            pltpu.VMEM((_CAND,), jnp.float32),
            pltpu.VMEM((_CAND,), jnp.float32),
            pltpu.VMEM((4 * _L,), jnp.float32),
            pltpu.VMEM((2 * _L,), jnp.int32),
            pltpu.VMEM((_OUTW,), jnp.float32),
            pltpu.VMEM((_OUTW,), jnp.float32),
            pltpu.SemaphoreType.DMA,
            pltpu.SemaphoreType.DMA,
        ],
    )
    def collect(x_hbm, t_hbm, outn_hbm, outp_hbm,
                xb0, tb0, xb1, tb1, cna, cnb, trk, cnts, oa, ob, sem0, sem1):
        ninf = jnp.full((_L,), _NEG_INF, jnp.float32)
        zc = jnp.zeros((_L,), jnp.int32)
        iot = lax.iota(jnp.int32, _L)
        wid = lax.axis_index("s") * nc + lax.axis_index("c")
        base = wid * slice_len

        def initb(j, _):
            cna[pl.ds(j * _L, _L)] = ninf
            cnb[pl.ds(j * _L, _L)] = ninf
            return 0

        lax.fori_loop(0, _CAND // _L, initb, 0)
        for j in range(4):
            trk[pl.ds(j * _L, _L)] = ninf
        cnts[pl.ds(0, _L)] = zc
        cnts[pl.ds(_L, _L)] = zc

        def merge_top32(off, gmd):
            hi = trk[pl.ds(off, _L)]
            nh = jnp.maximum(hi, gmd)
            sp = jnp.minimum(hi, gmd)
            nh, _ = plsc.sort_key_val(nh, nh, descending=False)
            spd, _ = plsc.sort_key_val(sp, sp, descending=True)
            lo_ = trk[pl.ds(off + _L, _L)]
            nl = jnp.maximum(lo_, spd)
            nl, _ = plsc.sort_key_val(nl, nl, descending=False)
            trk[pl.ds(off, _L)] = nh
            trk[pl.ds(off + _L, _L)] = nl

        def process_chunk(xb, tb):
            def sub_body(s, _):
                taua = trk[pl.ds(_L, _L)][0]
                taub = trk[pl.ds(3 * _L, _L)][0]
                ntaub = -taub

                def grp_body(g, __):
                    gbase = (s * 4 + g) * 8 * _L
                    xvs = [xb[pl.ds(gbase + j * _L, _L)] for j in range(8)]
                    mx0 = jnp.maximum(jnp.maximum(xvs[0], xvs[1]),
                                      jnp.maximum(xvs[2], xvs[3]))
                    mx1 = jnp.maximum(jnp.maximum(xvs[4], xvs[5]),
                                      jnp.maximum(xvs[6], xvs[7]))
                    mn0 = jnp.minimum(jnp.minimum(xvs[0], xvs[1]),
                                      jnp.minimum(xvs[2], xvs[3]))
                    mn1 = jnp.minimum(jnp.minimum(xvs[4], xvs[5]),
                                      jnp.minimum(xvs[6], xvs[7]))
                    mx = jnp.maximum(mx0, mx1)
                    mn = jnp.minimum(mn0, mn1)
                    hit = (mx > taua) | (mn < ntaub)
                    nhit = plsc.all_reduce_population_count(hit)[0]

                    @pl.when(nhit > 0)
                    def _():
                        cnta = cnts[pl.ds(0, _L)][0]
                        cntb = cnts[pl.ds(_L, _L)][0]
                        ga = ninf
                        gb = ninf
                        for j in range(8):
                            xv = xvs[j]
                            tv = tb[pl.ds(gbase + j * _L, _L)]
                            isneg = tv < 0.5
                            a = jnp.where(isneg, xv, ninf)
                            b = jnp.where(isneg, ninf, -xv)
                            ka = a > taua
                            kb = b > taub
                            ma = jnp.where(ka, a, ninf)
                            mb = jnp.where(kb, b, ninf)
                            sa, _ = plsc.sort_key_val(ma, ma, descending=True)
                            sb, _ = plsc.sort_key_val(mb, mb, descending=True)
                            cna[pl.ds(jnp.minimum(cnta, _CAND - _L), _L)] = sa
                            cnb[pl.ds(jnp.minimum(cntb, _CAND - _L), _L)] = sb
                            cnta = cnta + plsc.all_reduce_population_count(ka)[0]
                            cntb = cntb + plsc.all_reduce_population_count(kb)[0]
                            ga = jnp.maximum(ga, ma)
                            gb = jnp.maximum(gb, mb)
                        cnts[pl.ds(0, _L)] = zc + cnta
                        cnts[pl.ds(_L, _L)] = zc + cntb
                        gad, _ = plsc.sort_key_val(ga, ga, descending=True)
                        gbd, _ = plsc.sort_key_val(gb, gb, descending=True)
                        merge_top32(0, gad)
                        merge_top32(2 * _L, gbd)

                    return 0

                lax.fori_loop(0, 4, grp_body, 0)
                return 0

            lax.fori_loop(0, _SUBS, sub_body, 0)

        def mk(ref, c, buf, sem):
            return pltpu.make_async_copy(
                ref.at[pl.ds(base + c * _CHUNK, _CHUNK)], buf, sem)

        mk(x_hbm, 0, xb0, sem0).start()
        mk(t_hbm, 0, tb0, sem0).start()
        mk(x_hbm, 1, xb1, sem1).start()
        mk(t_hbm, 1, tb1, sem1).start()

        def pair_body(g, _):
            c0 = 2 * g
            c1 = 2 * g + 1
            mk(x_hbm, c0, xb0, sem0).wait()
            mk(t_hbm, c0, tb0, sem0).wait()
            process_chunk(xb0, tb0)

            @pl.when(c0 + 2 < chunks)
            def _():
                mk(x_hbm, c0 + 2, xb0, sem0).start()
                mk(t_hbm, c0 + 2, tb0, sem0).start()

            mk(x_hbm, c1, xb1, sem1).wait()
            mk(t_hbm, c1, tb1, sem1).wait()
            process_chunk(xb1, tb1)

            @pl.when(c1 + 2 < chunks)
            def _():
                mk(x_hbm, c1 + 2, xb1, sem1).start()
                mk(t_hbm, c1 + 2, tb1, sem1).start()

            return 0

        lax.fori_loop(0, chunks // 2, pair_body, 0)
        cnta = cnts[pl.ds(0, _L)][0]
        cntb = cnts[pl.ds(_L, _L)][0]

        def select25(cref, cnt):
            nv = (jnp.minimum(cnt, _CAND) + (_L - 1)) // _L

            def kbody(k, kc):
                filled, o0, o1 = kc

                def smax(j, m):
                    return jnp.maximum(m, cref[pl.ds(j * _L, _L)])

                v = jnp.max(lax.fori_loop(0, nv, smax, ninf))

                def srm(j, cacc):
                    vr = cref[pl.ds(j * _L, _L)]
                    eq = vr == v
                    cref[pl.ds(j * _L, _L)] = jnp.where(eq, ninf, vr)
                    return cacc + plsc.all_reduce_population_count(eq)

                cvec = lax.fori_loop(0, nv, srm, zc)
                take = jnp.minimum(cvec[0], _K - filled)
                lo = filled
                hi = filled + take
                o0 = jnp.where((iot >= lo) & (iot < hi), v, o0)
                o1 = jnp.where(((iot + _L) >= lo) & ((iot + _L) < hi), v, o1)
                return (filled + take, o0, o1)

            _, o0, o1 = lax.fori_loop(0, _K, kbody, (jnp.int32(0), ninf, ninf))
            return o0, o1

        o0, o1 = select25(cna, cnta)
        oa[pl.ds(0, _L)] = o0
        oa[pl.ds(_L, _L)] = o1
        p0, p1 = select25(cnb, cntb)
        ob[pl.ds(0, _L)] = p0
        ob[pl.ds(_L, _L)] = p1
        pltpu.sync_copy(oa, outn_hbm.at[pl.ds(wid * _OUTW, _OUTW)])
        pltpu.sync_copy(ob, outp_hbm.at[pl.ds(wid * _OUTW, _OUTW)])

    return collect


def _merge_body(nref, pref, oref):
    lane = lax.broadcasted_iota(jnp.int32, (1, 128), 1)

    def select25_tc(arr0):
        def kbody(k, kc):
            filled, out, arr = kc
            v = jnp.max(arr)
            eq = arr == v
            c = jnp.sum(eq.astype(jnp.int32))
            take = jnp.minimum(c, _K - filled)
            out = jnp.where((lane >= filled) & (lane < filled + take), v, out)
            arr = jnp.where(eq, _NEG_INF, arr)
            return (filled + take, out, arr)

        _, out, _ = lax.fori_loop(
            0, _K, kbody,
            (jnp.int32(0), jnp.full((1, 128), _NEG_INF, jnp.float32), arr0))
        return out

    m25 = lane < _K
    seln = select25_tc(nref[...])
    selp = select25_tc(pref[...])
    pn = jax.nn.sigmoid(seln)
    tn = jnp.maximum(jnp.log(1.0 - pn), -100.0)
    neg_loss = -0.5 * jnp.sum(jnp.where(m25, tn, 0.0)) / _K
    pp = jax.nn.sigmoid(-selp)
    tp = jnp.maximum(jnp.log(pp), -100.0)
    pos_loss = -0.5 * jnp.sum(jnp.where(m25, tp, 0.0)) / _K
    oref[...] = jnp.where(lane == 0, pos_loss,
                          jnp.where(lane == 1, neg_loss, 0.0))


def kernel(font_output, font_target, use_hard_mining):
    x = font_output.reshape(-1)
    t = font_target.reshape(-1)
    n = x.shape[0]

    def hard(_):
        negc, posc = _sc_collect(n)(x, t)
        out = pl.pallas_call(
            _merge_body,
            out_shape=jax.ShapeDtypeStruct((1, 128), jnp.float32),
        )(negc.reshape(8, -1), posc.reshape(8, -1))
        return out[0, 0], out[0, 1]

    def soft(_):
        p = jax.nn.sigmoid(x)
        pos_mask = t == 1
        neg_mask = t == 0
        logp = jnp.clip(jnp.log(p), -100.0, None)
        log1mp = jnp.clip(jnp.log(1.0 - p), -100.0, None)
        pos_loss = 0.5 * jnp.sum(jnp.where(pos_mask, -logp, 0.0)) / jnp.sum(pos_mask)
        neg_loss = 0.5 * jnp.sum(jnp.where(neg_mask, -log1mp, 0.0)) / jnp.sum(neg_mask)
        return pos_loss, neg_loss

    pos_loss, neg_loss = lax.cond(use_hard_mining != 0, hard, soft, operand=None)
    return (pos_loss + neg_loss, pos_loss, neg_loss)


# speedup vs baseline: 1.8210x; 1.2228x over previous
import functools

import jax
import jax.numpy as jnp
from jax import lax
from jax.experimental import pallas as pl
from jax.experimental.pallas import tpu as pltpu
from jax.experimental.pallas import tpu_sc as plsc

_L = 16
_K = 25
_NEG_INF = float("-inf")
_CHUNK = 8192
_SUBS = 16
_VPS = _CHUNK // (_SUBS * _L)
_CAND = 4096
_OUTW = 32


@functools.cache
def _sc_collect(rows, cols):
    info = plsc.get_sparse_core_info()
    nc, ns = info.num_cores, info.num_subcores
    nw = nc * ns
    slice_len = (rows * cols) // nw
    chunks = slice_len // _CHUNK
    cpr = cols // _CHUNK
    assert slice_len % _CHUNK == 0 and cols % _CHUNK == 0

    mesh = plsc.VectorSubcoreMesh(core_axis_name="c", subcore_axis_name="s")

    @functools.partial(
        pl.kernel,
        out_type=(
            jax.ShapeDtypeStruct((nw * _OUTW,), jnp.float32),
            jax.ShapeDtypeStruct((nw * _OUTW,), jnp.float32),
        ),
        mesh=mesh,
        compiler_params=pltpu.CompilerParams(needs_layout_passes=False),
        scratch_types=[
            pltpu.VMEM((_CHUNK,), jnp.float32),
            pltpu.VMEM((_CHUNK,), jnp.float32),
            pltpu.VMEM((_CHUNK,), jnp.float32),
            pltpu.VMEM((_CHUNK,), jnp.float32),
            pltpu.VMEM((_CAND,), jnp.float32),
            pltpu.VMEM((_CAND,), jnp.float32),
            pltpu.VMEM((4 * _L,), jnp.float32),
            pltpu.VMEM((2 * _L,), jnp.int32),
            pltpu.VMEM((_OUTW,), jnp.float32),
            pltpu.VMEM((_OUTW,), jnp.float32),
            pltpu.SemaphoreType.DMA,
            pltpu.SemaphoreType.DMA,
        ],
    )
    def collect(x_hbm, t_hbm, outn_hbm, outp_hbm,
                xb0, tb0, xb1, tb1, cna, cnb, trk, cnts, oa, ob, sem0, sem1):
        ninf = jnp.full((_L,), _NEG_INF, jnp.float32)
        zc = jnp.zeros((_L,), jnp.int32)
        iot = lax.iota(jnp.int32, _L)
        wid = lax.axis_index("s") * nc + lax.axis_index("c")

        def initb(j, _):
            cna[pl.ds(j * _L, _L)] = ninf
            cnb[pl.ds(j * _L, _L)] = ninf
            return 0

        lax.fori_loop(0, _CAND // _L, initb, 0)
        for j in range(4):
            trk[pl.ds(j * _L, _L)] = ninf
        cnts[pl.ds(0, _L)] = zc
        cnts[pl.ds(_L, _L)] = zc

        def merge_top32(off, gmd):
            hi = trk[pl.ds(off, _L)]
            nh = jnp.maximum(hi, gmd)
            sp = jnp.minimum(hi, gmd)
            nh, _ = plsc.sort_key_val(nh, nh, descending=False)
            spd, _ = plsc.sort_key_val(sp, sp, descending=True)
            lo_ = trk[pl.ds(off + _L, _L)]
            nl = jnp.maximum(lo_, spd)
            nl, _ = plsc.sort_key_val(nl, nl, descending=False)
            trk[pl.ds(off, _L)] = nh
            trk[pl.ds(off + _L, _L)] = nl

        def process_chunk(xb, tb):
            def sub_body(s, _):
                taua = trk[pl.ds(_L, _L)][0]
                taub = trk[pl.ds(3 * _L, _L)][0]
                ntaub = -taub

                def grp_body(g, __):
                    gbase = (s * 4 + g) * 8 * _L
                    xvs = [xb[pl.ds(gbase + j * _L, _L)] for j in range(8)]
                    mx0 = jnp.maximum(jnp.maximum(xvs[0], xvs[1]),
                                      jnp.maximum(xvs[2], xvs[3]))
                    mx1 = jnp.maximum(jnp.maximum(xvs[4], xvs[5]),
                                      jnp.maximum(xvs[6], xvs[7]))
                    mn0 = jnp.minimum(jnp.minimum(xvs[0], xvs[1]),
                                      jnp.minimum(xvs[2], xvs[3]))
                    mn1 = jnp.minimum(jnp.minimum(xvs[4], xvs[5]),
                                      jnp.minimum(xvs[6], xvs[7]))
                    mx = jnp.maximum(mx0, mx1)
                    mn = jnp.minimum(mn0, mn1)
                    hit = (mx > taua) | (mn < ntaub)
                    nhit = plsc.all_reduce_population_count(hit)[0]

                    @pl.when(nhit > 0)
                    def _():
                        cnta = cnts[pl.ds(0, _L)][0]
                        cntb = cnts[pl.ds(_L, _L)][0]
                        ga = ninf
                        gb = ninf
                        for j in range(8):
                            xv = xvs[j]
                            tv = tb[pl.ds(gbase + j * _L, _L)]
                            isneg = tv < 0.5
                            a = jnp.where(isneg, xv, ninf)
                            b = jnp.where(isneg, ninf, -xv)
                            ka = a > taua
                            kb = b > taub
                            ma = jnp.where(ka, a, ninf)
                            mb = jnp.where(kb, b, ninf)
                            sa, _ = plsc.sort_key_val(ma, ma, descending=True)
                            sb, _ = plsc.sort_key_val(mb, mb, descending=True)
                            cna[pl.ds(jnp.minimum(cnta, _CAND - _L), _L)] = sa
                            cnb[pl.ds(jnp.minimum(cntb, _CAND - _L), _L)] = sb
                            cnta = cnta + plsc.all_reduce_population_count(ka)[0]
                            cntb = cntb + plsc.all_reduce_population_count(kb)[0]
                            ga = jnp.maximum(ga, ma)
                            gb = jnp.maximum(gb, mb)
                        cnts[pl.ds(0, _L)] = zc + cnta
                        cnts[pl.ds(_L, _L)] = zc + cntb
                        gad, _ = plsc.sort_key_val(ga, ga, descending=True)
                        gbd, _ = plsc.sort_key_val(gb, gb, descending=True)
                        merge_top32(0, gad)
                        merge_top32(2 * _L, gbd)

                    return 0

                lax.fori_loop(0, 4, grp_body, 0)
                return 0

            lax.fori_loop(0, _SUBS, sub_body, 0)

        def mk(ref, c, buf, sem):
            gc = wid * chunks + c
            return pltpu.make_async_copy(
                ref.at[gc // cpr, pl.ds((gc % cpr) * _CHUNK, _CHUNK)],
                buf, sem)

        mk(x_hbm, 0, xb0, sem0).start()
        mk(t_hbm, 0, tb0, sem0).start()
        mk(x_hbm, 1, xb1, sem1).start()
        mk(t_hbm, 1, tb1, sem1).start()

        def pair_body(g, _):
            c0 = 2 * g
            c1 = 2 * g + 1
            mk(x_hbm, c0, xb0, sem0).wait()
            mk(t_hbm, c0, tb0, sem0).wait()
            process_chunk(xb0, tb0)

            @pl.when(c0 + 2 < chunks)
            def _():
                mk(x_hbm, c0 + 2, xb0, sem0).start()
                mk(t_hbm, c0 + 2, tb0, sem0).start()

            mk(x_hbm, c1, xb1, sem1).wait()
            mk(t_hbm, c1, tb1, sem1).wait()
            process_chunk(xb1, tb1)

            @pl.when(c1 + 2 < chunks)
            def _():
                mk(x_hbm, c1 + 2, xb1, sem1).start()
                mk(t_hbm, c1 + 2, tb1, sem1).start()

            return 0

        lax.fori_loop(0, chunks // 2, pair_body, 0)
        cnta = cnts[pl.ds(0, _L)][0]
        cntb = cnts[pl.ds(_L, _L)][0]

        def select25(cref, cnt):
            nv = (jnp.minimum(cnt, _CAND) + (_L - 1)) // _L

            def kbody(k, kc):
                filled, o0, o1 = kc

                def smax(j, m):
                    return jnp.maximum(m, cref[pl.ds(j * _L, _L)])

                v = jnp.max(lax.fori_loop(0, nv, smax, ninf))

                def srm(j, cacc):
                    vr = cref[pl.ds(j * _L, _L)]
                    eq = vr == v
                    cref[pl.ds(j * _L, _L)] = jnp.where(eq, ninf, vr)
                    return cacc + plsc.all_reduce_population_count(eq)

                cvec = lax.fori_loop(0, nv, srm, zc)
                take = jnp.minimum(cvec[0], _K - filled)
                lo = filled
                hi = filled + take
                o0 = jnp.where((iot >= lo) & (iot < hi), v, o0)
                o1 = jnp.where(((iot + _L) >= lo) & ((iot + _L) < hi), v, o1)
                return (filled + take, o0, o1)

            _, o0, o1 = lax.fori_loop(0, _K, kbody, (jnp.int32(0), ninf, ninf))
            return o0, o1

        o0, o1 = select25(cna, cnta)
        oa[pl.ds(0, _L)] = o0
        oa[pl.ds(_L, _L)] = o1
        p0, p1 = select25(cnb, cntb)
        ob[pl.ds(0, _L)] = p0
        ob[pl.ds(_L, _L)] = p1
        pltpu.sync_copy(oa, outn_hbm.at[pl.ds(wid * _OUTW, _OUTW)])
        pltpu.sync_copy(ob, outp_hbm.at[pl.ds(wid * _OUTW, _OUTW)])

    return collect


def _merge_body(nref, pref, oref):
    lane = lax.broadcasted_iota(jnp.int32, (1, 128), 1)

    def select25_tc(arr0):
        def kbody(k, kc):
            filled, out, arr = kc
            v = jnp.max(arr)
            eq = arr == v
            c = jnp.sum(eq.astype(jnp.int32))
            take = jnp.minimum(c, _K - filled)
            out = jnp.where((lane >= filled) & (lane < filled + take), v, out)
            arr = jnp.where(eq, _NEG_INF, arr)
            return (filled + take, out, arr)

        _, out, _ = lax.fori_loop(
            0, _K, kbody,
            (jnp.int32(0), jnp.full((1, 128), _NEG_INF, jnp.float32), arr0))
        return out

    m25 = lane < _K
    seln = select25_tc(nref[...])
    selp = select25_tc(pref[...])
    pn = jax.nn.sigmoid(seln)
    tn = jnp.maximum(jnp.log(1.0 - pn), -100.0)
    neg_loss = -0.5 * jnp.sum(jnp.where(m25, tn, 0.0)) / _K
    pp = jax.nn.sigmoid(-selp)
    tp = jnp.maximum(jnp.log(pp), -100.0)
    pos_loss = -0.5 * jnp.sum(jnp.where(m25, tp, 0.0)) / _K
    oref[...] = jnp.where(lane == 0, pos_loss,
                          jnp.where(lane == 1, neg_loss, 0.0))


def kernel(font_output, font_target, use_hard_mining):
    x = font_output
    t = font_target
    rows, cols = x.shape

    def hard(_):
        negc, posc = _sc_collect(rows, cols)(x, t)
        out = pl.pallas_call(
            _merge_body,
            out_shape=jax.ShapeDtypeStruct((1, 128), jnp.float32),
        )(negc.reshape(8, -1), posc.reshape(8, -1))
        return out[0, 0], out[0, 1]

    def soft(_):
        p = jax.nn.sigmoid(x)
        pos_mask = t == 1
        neg_mask = t == 0
        logp = jnp.clip(jnp.log(p), -100.0, None)
        log1mp = jnp.clip(jnp.log(1.0 - p), -100.0, None)
        pos_loss = 0.5 * jnp.sum(jnp.where(pos_mask, -logp, 0.0)) / jnp.sum(pos_mask)
        neg_loss = 0.5 * jnp.sum(jnp.where(neg_mask, -log1mp, 0.0)) / jnp.sum(neg_mask)
        return pos_loss, neg_loss

    pos_loss, neg_loss = lax.cond(use_hard_mining != 0, hard, soft, operand=None)
    return (pos_loss + neg_loss, pos_loss, neg_loss)
